# baseline (speedup 1.0000x reference)
import jax
import jax.numpy as jnp
from jax import lax
from jax.experimental import pallas as pl
from jax.experimental.pallas import tpu as pltpu
from jax.experimental.pallas import tpu_sc as plsc

N_NODES = 10000
N_EDGES = 320000
D_NODE = 128
D_EDGE = 16
HIDDEN = 128

NC = 2
NS = 16
L = 16
NW = NC * NS
CHUNK = 80
EPW = N_EDGES // NW
CPW = EPW // CHUNK
N_PAD = 10240
ROWS_PER_TILE = N_PAD // NS
ZROWS = 128

_sc_mesh = plsc.VectorSubcoreMesh(core_axis_name="c", subcore_axis_name="s",
                                  num_cores=NC, num_subcores=NS)


def _sc_edge_body(send_hbm, recv_hbm, p_hbm, eproj_hbm, a_hbm,
                  ue_hbm, agg_hbm,
                  idx_s0, idx_s1, idx_r0, idx_r1,
                  gbuf, pbuf, a_v, agg_sh, sem_g, sem_o):
    c = lax.axis_index("c")
    s = lax.axis_index("s")
    w = s * NC + c

    idx_s = (idx_s0, idx_s1)
    idx_r = (idx_r0, idx_r1)

    @pl.loop(0, CHUNK)
    def _zero_row(t):
        zero = jnp.zeros((L,), jnp.float32)
        for c8 in range(HIDDEN // L):
            gbuf[t, pl.ds(c8 * L, L)] = zero

    row0 = pl.multiple_of(s * ROWS_PER_TILE, 8)
    for rep in range(ROWS_PER_TILE // CHUNK):
        pltpu.sync_copy(gbuf, agg_sh.at[pl.ds(row0 + rep * CHUNK, CHUNK)])
    pltpu.sync_copy(a_hbm, a_v)
    plsc.subcore_barrier()

    def ebase(j):
        return pl.multiple_of(w * EPW + j * CHUNK, 8)

    def load_idx(j, b):
        eb = ebase(j)
        pltpu.sync_copy(send_hbm.at[pl.ds(eb, CHUNK)], idx_s[b])
        pltpu.sync_copy(recv_hbm.at[pl.ds(eb, CHUNK)], idx_r[b])

    avs = tuple(a_v[pl.ds(c8 * L, L)] for c8 in range(HIDDEN // L))

    def compute():
        @pl.loop(0, CHUNK, unroll=4)
        def _row(t):
            for c8 in range(HIDDEN // L):
                sl = pl.ds(c8 * L, L)
                v = gbuf[t, sl] + pbuf[t, sl]
                pbuf[t, sl] = (jnp.maximum(v, 0.0)
                               + avs[c8] * jnp.minimum(v, 0.0))

    def body(j, b, prefetch_next=True):
        eb = ebase(j)
        desc_g = pltpu.async_copy(p_hbm.at[idx_s[b]], gbuf, sem_g)
        pltpu.sync_copy(eproj_hbm.at[pl.ds(eb, CHUNK)], pbuf)
        if prefetch_next:
            load_idx(j + 1, (b + 1) % 2)
        desc_g.wait()
        compute()
        desc_s = pltpu.async_copy(pbuf, ue_hbm.at[pl.ds(eb, CHUNK)], sem_o)
        pltpu.sync_copy(pbuf, agg_sh.at[idx_r[b]], add=True)
        desc_s.wait()

    load_idx(0, 0)

    @pl.loop(0, CPW - 1, step=2)
    def _pair(j0):
        body(j0, 0)
        body(j0 + 1, 1)

    body(CPW - 1, 0, prefetch_next=False)

    plsc.subcore_barrier()
    pltpu.sync_copy(agg_sh.at[pl.ds(row0, ROWS_PER_TILE)],
                    agg_hbm.at[c, pl.ds(row0, ROWS_PER_TILE)])


_sc_edge = pl.kernel(
    _sc_edge_body,
    out_type=(
        jax.ShapeDtypeStruct((N_EDGES, HIDDEN), jnp.float32),
        jax.ShapeDtypeStruct((NC, N_PAD, HIDDEN), jnp.float32),
    ),
    mesh=_sc_mesh,
    scratch_types=(
        [pltpu.VMEM((CHUNK,), jnp.int32)] * 4
        + [pltpu.VMEM((CHUNK, HIDDEN), jnp.float32)] * 2
        + [pltpu.VMEM((HIDDEN,), jnp.float32),
           pltpu.VMEM_SHARED((N_PAD, HIDDEN), jnp.float32)]
        + [pltpu.SemaphoreType.DMA] * 2
    ),
)


def _matmul_body(x_ref, w_ref, o_ref):
    o_ref[...] = jnp.dot(x_ref[...], w_ref[...],
                         preferred_element_type=jnp.float32)


def _node_proj(x, w_t):
    blk = 2000
    return pl.pallas_call(
        _matmul_body,
        grid=(N_NODES // blk,),
        in_specs=[pl.BlockSpec((blk, D_NODE), lambda i: (i, 0)),
                  pl.BlockSpec((D_NODE, HIDDEN), lambda i: (0, 0))],
        out_specs=pl.BlockSpec((blk, HIDDEN), lambda i: (i, 0)),
        out_shape=jax.ShapeDtypeStruct((N_NODES, HIDDEN), jnp.float32),
    )(x, w_t)


def _edge_proj(ef, w_t):
    blk = 4000
    return pl.pallas_call(
        _matmul_body,
        grid=(N_EDGES // blk,),
        in_specs=[pl.BlockSpec((blk, D_EDGE), lambda i: (i, 0)),
                  pl.BlockSpec((D_EDGE, HIDDEN), lambda i: (0, 0))],
        out_specs=pl.BlockSpec((blk, HIDDEN), lambda i: (i, 0)),
        out_shape=jax.ShapeDtypeStruct((N_EDGES, HIDDEN), jnp.float32),
    )(ef, w_t)


def _node_mlp_body(a0_ref, a1_ref, x_ref, wa_ref, wx_ref, an_ref, o_ref):
    acc = jnp.dot(a0_ref[...] + a1_ref[...], wa_ref[...],
                  preferred_element_type=jnp.float32)
    acc = acc + jnp.dot(x_ref[...], wx_ref[...],
                        preferred_element_type=jnp.float32)
    a = an_ref[...]
    o_ref[...] = jnp.where(acc >= 0.0, acc, acc * a)


def _node_mlp(agg0, agg1, x, wa_t, wx_t, a_n):
    blk = 2000
    return pl.pallas_call(
        _node_mlp_body,
        grid=(N_NODES // blk,),
        in_specs=[pl.BlockSpec((blk, HIDDEN), lambda i: (i, 0)),
                  pl.BlockSpec((blk, HIDDEN), lambda i: (i, 0)),
                  pl.BlockSpec((blk, D_NODE), lambda i: (i, 0)),
                  pl.BlockSpec((HIDDEN, HIDDEN), lambda i: (0, 0)),
                  pl.BlockSpec((D_NODE, HIDDEN), lambda i: (0, 0)),
                  pl.BlockSpec((1, HIDDEN), lambda i: (0, 0))],
        out_specs=pl.BlockSpec((blk, HIDDEN), lambda i: (i, 0)),
        out_shape=jax.ShapeDtypeStruct((N_NODES, HIDDEN), jnp.float32),
    )(agg0, agg1, x, wa_t, wx_t, a_n)


def kernel(node_features, edge_index, edge_features, W_e, a_e, W_n, a_n):
    receivers = edge_index[0]
    senders = edge_index[1]
    w_en_t = W_e[:, :D_NODE].T
    w_ee_t = W_e[:, D_NODE:].T
    w_na_t = W_n[:, :HIDDEN].T
    w_nx_t = W_n[:, HIDDEN:].T

    p = _node_proj(node_features, w_en_t)
    eproj = _edge_proj(edge_features, w_ee_t)

    updated_edge_features, agg_parts = _sc_edge(
        senders, receivers, p, eproj, a_e)

    updated_node_features = _node_mlp(
        agg_parts[0, :N_NODES], agg_parts[1, :N_NODES], node_features,
        w_na_t, w_nx_t, a_n.reshape(1, HIDDEN))
    return (updated_node_features, updated_edge_features)


# speedup vs baseline: 1.6916x; 1.6916x over previous
import jax
import jax.numpy as jnp
from jax import lax
from jax.experimental import pallas as pl
from jax.experimental.pallas import tpu as pltpu
from jax.experimental.pallas import tpu_sc as plsc

N_NODES = 10000
N_EDGES = 320000
D_NODE = 128
D_EDGE = 16
HIDDEN = 128

NC = 2
NS = 16
L = 16
NW = NC * NS
CHUNK = 80
EPW = N_EDGES // NW
CPW = EPW // CHUNK
N_PAD = 10240
ROWS_PER_TILE = N_PAD // NS
ZROWS = 128

_sc_mesh = plsc.VectorSubcoreMesh(core_axis_name="c", subcore_axis_name="s",
                                  num_cores=NC, num_subcores=NS)


def _sc_edge_body(send_hbm, recv_hbm, p_hbm, eproj_hbm, a_hbm,
                  ue_hbm, agg_hbm,
                  idx_s0, idx_s1, idx_r0, idx_r1,
                  gbuf, pbuf, a_v, agg_sh, sem_g, sem_o):
    c = lax.axis_index("c")
    s = lax.axis_index("s")
    w = s * NC + c

    idx_s = (idx_s0, idx_s1)
    idx_r = (idx_r0, idx_r1)

    @pl.loop(0, CHUNK)
    def _zero_row(t):
        zero = jnp.zeros((L,), jnp.float32)
        for c8 in range(HIDDEN // L):
            gbuf[t, pl.ds(c8 * L, L)] = zero

    row0 = pl.multiple_of(s * ROWS_PER_TILE, 8)
    for rep in range(ROWS_PER_TILE // CHUNK):
        pltpu.sync_copy(gbuf, agg_sh.at[pl.ds(row0 + rep * CHUNK, CHUNK)])
    pltpu.sync_copy(a_hbm, a_v)
    plsc.subcore_barrier()

    def ebase(j):
        return pl.multiple_of(w * EPW + j * CHUNK, 8)

    def load_idx(j, b):
        eb = ebase(j)
        pltpu.sync_copy(send_hbm.at[pl.ds(eb, CHUNK)], idx_s[b])
        pltpu.sync_copy(recv_hbm.at[pl.ds(eb, CHUNK)], idx_r[b])

    avs = tuple(a_v[pl.ds(c8 * L, L)] for c8 in range(HIDDEN // L))

    def compute():
        @pl.loop(0, CHUNK)
        def _row(t):
            for c8 in range(HIDDEN // L):
                sl = pl.ds(c8 * L, L)
                v = gbuf[t, sl] + pbuf[t, sl]
                pbuf[t, sl] = jnp.where(v >= 0.0, v, v * avs[c8])

    def body(j, b, prefetch_next=True):
        eb = ebase(j)
        desc_g = pltpu.async_copy(p_hbm.at[idx_s[b]], gbuf, sem_g)
        pltpu.sync_copy(eproj_hbm.at[pl.ds(eb, CHUNK)], pbuf)
        if prefetch_next:
            load_idx(j + 1, (b + 1) % 2)
        desc_g.wait()
        compute()
        desc_s = pltpu.async_copy(pbuf, ue_hbm.at[pl.ds(eb, CHUNK)], sem_o)
        pltpu.sync_copy(pbuf, agg_sh.at[idx_r[b]], add=True)
        desc_s.wait()

    load_idx(0, 0)

    @pl.loop(0, CPW - 1, step=2)
    def _pair(j0):
        body(j0, 0)
        body(j0 + 1, 1)

    body(CPW - 1, 0, prefetch_next=False)

    plsc.subcore_barrier()
    pltpu.sync_copy(agg_sh.at[pl.ds(row0, ROWS_PER_TILE)],
                    agg_hbm.at[c, pl.ds(row0, ROWS_PER_TILE)])


_sc_edge = pl.kernel(
    _sc_edge_body,
    out_type=(
        jax.ShapeDtypeStruct((N_EDGES, HIDDEN), jnp.float32),
        jax.ShapeDtypeStruct((NC, N_PAD, HIDDEN), jnp.float32),
    ),
    mesh=_sc_mesh,
    scratch_types=(
        [pltpu.VMEM((CHUNK,), jnp.int32)] * 4
        + [pltpu.VMEM((CHUNK, HIDDEN), jnp.float32)] * 2
        + [pltpu.VMEM((HIDDEN,), jnp.float32),
           pltpu.VMEM_SHARED((N_PAD, HIDDEN), jnp.float32)]
        + [pltpu.SemaphoreType.DMA] * 2
    ),
)


def _matmul_body(x_ref, w_ref, o_ref):
    o_ref[...] = jnp.dot(x_ref[...], w_ref[...],
                         preferred_element_type=jnp.float32)


def _node_proj(x, w_t):
    blk = 2000
    return pl.pallas_call(
        _matmul_body,
        grid=(N_NODES // blk,),
        in_specs=[pl.BlockSpec((blk, D_NODE), lambda i: (i, 0)),
                  pl.BlockSpec((D_NODE, HIDDEN), lambda i: (0, 0))],
        out_specs=pl.BlockSpec((blk, HIDDEN), lambda i: (i, 0)),
        out_shape=jax.ShapeDtypeStruct((N_NODES, HIDDEN), jnp.float32),
    )(x, w_t)


def _edge_proj(ef, w_t):
    blk = 4000
    return pl.pallas_call(
        _matmul_body,
        grid=(N_EDGES // blk,),
        in_specs=[pl.BlockSpec((blk, D_EDGE), lambda i: (i, 0)),
                  pl.BlockSpec((D_EDGE, HIDDEN), lambda i: (0, 0))],
        out_specs=pl.BlockSpec((blk, HIDDEN), lambda i: (i, 0)),
        out_shape=jax.ShapeDtypeStruct((N_EDGES, HIDDEN), jnp.float32),
    )(ef, w_t)


def _node_mlp_body(a0_ref, a1_ref, x_ref, wa_ref, wx_ref, an_ref, o_ref):
    acc = jnp.dot(a0_ref[...] + a1_ref[...], wa_ref[...],
                  preferred_element_type=jnp.float32)
    acc = acc + jnp.dot(x_ref[...], wx_ref[...],
                        preferred_element_type=jnp.float32)
    a = an_ref[...]
    o_ref[...] = jnp.where(acc >= 0.0, acc, acc * a)


def _node_mlp(agg0, agg1, x, wa_t, wx_t, a_n):
    blk = 2000
    return pl.pallas_call(
        _node_mlp_body,
        grid=(N_NODES // blk,),
        in_specs=[pl.BlockSpec((blk, HIDDEN), lambda i: (i, 0)),
                  pl.BlockSpec((blk, HIDDEN), lambda i: (i, 0)),
                  pl.BlockSpec((blk, D_NODE), lambda i: (i, 0)),
                  pl.BlockSpec((HIDDEN, HIDDEN), lambda i: (0, 0)),
                  pl.BlockSpec((D_NODE, HIDDEN), lambda i: (0, 0)),
                  pl.BlockSpec((1, HIDDEN), lambda i: (0, 0))],
        out_specs=pl.BlockSpec((blk, HIDDEN), lambda i: (i, 0)),
        out_shape=jax.ShapeDtypeStruct((N_NODES, HIDDEN), jnp.float32),
    )(agg0, agg1, x, wa_t, wx_t, a_n)


def kernel(node_features, edge_index, edge_features, W_e, a_e, W_n, a_n):
    receivers = edge_index[0]
    senders = edge_index[1]
    w_en_t = W_e[:, :D_NODE].T
    w_ee_t = W_e[:, D_NODE:].T
    w_na_t = W_n[:, :HIDDEN].T
    w_nx_t = W_n[:, HIDDEN:].T

    p = _node_proj(node_features, w_en_t)
    eproj = _edge_proj(edge_features, w_ee_t)

    updated_edge_features, agg_parts = _sc_edge(
        senders, receivers, p, eproj, a_e)

    updated_node_features = _node_mlp(
        agg_parts[0, :N_NODES], agg_parts[1, :N_NODES], node_features,
        w_na_t, w_nx_t, a_n.reshape(1, HIDDEN))
    return (updated_node_features, updated_edge_features)


# speedup vs baseline: 1.9647x; 1.1614x over previous
import jax
import jax.numpy as jnp
from jax import lax
from jax.experimental import pallas as pl
from jax.experimental.pallas import tpu as pltpu
from jax.experimental.pallas import tpu_sc as plsc

N_NODES = 10000
N_EDGES = 320000
D_NODE = 128
D_EDGE = 16
HIDDEN = 128

NC = 2
NS = 16
L = 16
NW = NC * NS
CHUNK = 80
EPW = N_EDGES // NW
CPW = EPW // CHUNK
N_PAD = 10240
ROWS_PER_TILE = N_PAD // NS
ZROWS = 128

_sc_mesh = plsc.VectorSubcoreMesh(core_axis_name="c", subcore_axis_name="s",
                                  num_cores=NC, num_subcores=NS)


def _sc_edge_body(send_hbm, recv_hbm, p_hbm, eproj_hbm, a_hbm,
                  ue_hbm, agg_hbm,
                  idx_s0, idx_s1, idx_r0, idx_r1,
                  gbuf, pbuf, a_v, agg_sh, sem_g, sem_e, sem_o):
    c = lax.axis_index("c")
    s = lax.axis_index("s")
    w = s * NC + c

    idx_s = (idx_s0, idx_s1)
    idx_r = (idx_r0, idx_r1)

    @pl.loop(0, CHUNK)
    def _zero_row(t):
        zero = jnp.zeros((L,), jnp.float32)
        for c8 in range(HIDDEN // L):
            gbuf[t, pl.ds(c8 * L, L)] = zero

    row0 = pl.multiple_of(s * ROWS_PER_TILE, 8)
    for rep in range(ROWS_PER_TILE // CHUNK):
        pltpu.sync_copy(gbuf, agg_sh.at[pl.ds(row0 + rep * CHUNK, CHUNK)])
    pltpu.sync_copy(a_hbm, a_v)
    plsc.subcore_barrier()

    def ebase(j):
        return pl.multiple_of(w * EPW + j * CHUNK, 8)

    def load_idx(j, b):
        eb = ebase(j)
        pltpu.sync_copy(send_hbm.at[pl.ds(eb, CHUNK)], idx_s[b])
        pltpu.sync_copy(recv_hbm.at[pl.ds(eb, CHUNK)], idx_r[b])

    avs = tuple(a_v[pl.ds(c8 * L, L)] for c8 in range(HIDDEN // L))

    def compute():
        @pl.loop(0, CHUNK)
        def _row(t):
            for c8 in range(HIDDEN // L):
                sl = pl.ds(c8 * L, L)
                v = gbuf[t, sl] + pbuf[t, sl]
                pbuf[t, sl] = jnp.where(v >= 0.0, v, v * avs[c8])

    def body(j, b, prefetch_next=True):
        eb = ebase(j)
        desc_g = pltpu.async_copy(p_hbm.at[idx_s[b]], gbuf, sem_g)
        desc_e = pltpu.async_copy(eproj_hbm.at[pl.ds(eb, CHUNK)], pbuf, sem_e)
        if prefetch_next:
            load_idx(j + 1, (b + 1) % 2)
        desc_e.wait()
        desc_g.wait()
        compute()
        desc_s = pltpu.async_copy(pbuf, ue_hbm.at[pl.ds(eb, CHUNK)], sem_o)
        pltpu.sync_copy(pbuf, agg_sh.at[idx_r[b]], add=True)
        desc_s.wait()

    load_idx(0, 0)

    @pl.loop(0, CPW - 1, step=2)
    def _pair(j0):
        body(j0, 0)
        body(j0 + 1, 1)

    body(CPW - 1, 0, prefetch_next=False)

    plsc.subcore_barrier()
    pltpu.sync_copy(agg_sh.at[pl.ds(row0, ROWS_PER_TILE)],
                    agg_hbm.at[c, pl.ds(row0, ROWS_PER_TILE)])


_sc_edge = pl.kernel(
    _sc_edge_body,
    out_type=(
        jax.ShapeDtypeStruct((N_EDGES, HIDDEN), jnp.float32),
        jax.ShapeDtypeStruct((NC, N_PAD, HIDDEN), jnp.float32),
    ),
    mesh=_sc_mesh,
    scratch_types=(
        [pltpu.VMEM((CHUNK,), jnp.int32)] * 4
        + [pltpu.VMEM((CHUNK, HIDDEN), jnp.float32)] * 2
        + [pltpu.VMEM((HIDDEN,), jnp.float32),
           pltpu.VMEM_SHARED((N_PAD, HIDDEN), jnp.float32)]
        + [pltpu.SemaphoreType.DMA] * 3
    ),
)


def _matmul_body(x_ref, w_ref, o_ref):
    o_ref[...] = jnp.dot(x_ref[...], w_ref[...],
                         preferred_element_type=jnp.float32)


def _node_proj(x, w_t):
    blk = 2000
    return pl.pallas_call(
        _matmul_body,
        grid=(N_NODES // blk,),
        in_specs=[pl.BlockSpec((blk, D_NODE), lambda i: (i, 0)),
                  pl.BlockSpec((D_NODE, HIDDEN), lambda i: (0, 0))],
        out_specs=pl.BlockSpec((blk, HIDDEN), lambda i: (i, 0)),
        out_shape=jax.ShapeDtypeStruct((N_NODES, HIDDEN), jnp.float32),
    )(x, w_t)


def _edge_proj(ef, w_t):
    blk = 4000
    return pl.pallas_call(
        _matmul_body,
        grid=(N_EDGES // blk,),
        in_specs=[pl.BlockSpec((blk, D_EDGE), lambda i: (i, 0)),
                  pl.BlockSpec((D_EDGE, HIDDEN), lambda i: (0, 0))],
        out_specs=pl.BlockSpec((blk, HIDDEN), lambda i: (i, 0)),
        out_shape=jax.ShapeDtypeStruct((N_EDGES, HIDDEN), jnp.float32),
    )(ef, w_t)


def _node_mlp_body(a0_ref, a1_ref, x_ref, wa_ref, wx_ref, an_ref, o_ref):
    acc = jnp.dot(a0_ref[...] + a1_ref[...], wa_ref[...],
                  preferred_element_type=jnp.float32)
    acc = acc + jnp.dot(x_ref[...], wx_ref[...],
                        preferred_element_type=jnp.float32)
    a = an_ref[...]
    o_ref[...] = jnp.where(acc >= 0.0, acc, acc * a)


def _node_mlp(agg0, agg1, x, wa_t, wx_t, a_n):
    blk = 2000
    return pl.pallas_call(
        _node_mlp_body,
        grid=(N_NODES // blk,),
        in_specs=[pl.BlockSpec((blk, HIDDEN), lambda i: (i, 0)),
                  pl.BlockSpec((blk, HIDDEN), lambda i: (i, 0)),
                  pl.BlockSpec((blk, D_NODE), lambda i: (i, 0)),
                  pl.BlockSpec((HIDDEN, HIDDEN), lambda i: (0, 0)),
                  pl.BlockSpec((D_NODE, HIDDEN), lambda i: (0, 0)),
                  pl.BlockSpec((1, HIDDEN), lambda i: (0, 0))],
        out_specs=pl.BlockSpec((blk, HIDDEN), lambda i: (i, 0)),
        out_shape=jax.ShapeDtypeStruct((N_NODES, HIDDEN), jnp.float32),
    )(agg0, agg1, x, wa_t, wx_t, a_n)


def kernel(node_features, edge_index, edge_features, W_e, a_e, W_n, a_n):
    receivers = edge_index[0]
    senders = edge_index[1]
    w_en_t = W_e[:, :D_NODE].T
    w_ee_t = W_e[:, D_NODE:].T
    w_na_t = W_n[:, :HIDDEN].T
    w_nx_t = W_n[:, HIDDEN:].T

    p = _node_proj(node_features, w_en_t)
    eproj = _edge_proj(edge_features, w_ee_t)

    updated_edge_features, agg_parts = _sc_edge(
        senders, receivers, p, eproj, a_e)

    updated_node_features = _node_mlp(
        agg_parts[0, :N_NODES], agg_parts[1, :N_NODES], node_features,
        w_na_t, w_nx_t, a_n.reshape(1, HIDDEN))
    return (updated_node_features, updated_edge_features)


# speedup vs baseline: 2.1916x; 1.1155x over previous
import jax
import jax.numpy as jnp
from jax import lax
from jax.experimental import pallas as pl
from jax.experimental.pallas import tpu as pltpu
from jax.experimental.pallas import tpu_sc as plsc

N_NODES = 10000
N_EDGES = 320000
D_NODE = 128
D_EDGE = 16
HIDDEN = 128

NC = 2
NS = 16
L = 16
NW = NC * NS
CHUNK = 80
EPW = N_EDGES // NW
CPW = EPW // CHUNK
N_PAD = 10240
ROWS_PER_TILE = N_PAD // NS
ZROWS = 128

_sc_mesh = plsc.VectorSubcoreMesh(core_axis_name="c", subcore_axis_name="s",
                                  num_cores=NC, num_subcores=NS)


def _sc_edge_body(send_hbm, recv_hbm, p_hbm, eproj_hbm, a_hbm,
                  ue_hbm, agg_hbm,
                  idx_s0, idx_s1, idx_r0, idx_r1,
                  gbuf0, gbuf1, pbuf0, pbuf1, a_v, agg_sh,
                  sem_g0, sem_g1, sem_e0, sem_e1, sem_o0, sem_o1):
    c = lax.axis_index("c")
    s = lax.axis_index("s")
    w = s * NC + c

    idx_s = (idx_s0, idx_s1)
    idx_r = (idx_r0, idx_r1)
    gbuf = (gbuf0, gbuf1)
    pbuf = (pbuf0, pbuf1)
    sem_g = (sem_g0, sem_g1)
    sem_e = (sem_e0, sem_e1)
    sem_o = (sem_o0, sem_o1)

    @pl.loop(0, CHUNK)
    def _zero_row(t):
        zero = jnp.zeros((L,), jnp.float32)
        for c8 in range(HIDDEN // L):
            gbuf0[t, pl.ds(c8 * L, L)] = zero

    row0 = pl.multiple_of(s * ROWS_PER_TILE, 8)
    for rep in range(ROWS_PER_TILE // CHUNK):
        pltpu.sync_copy(gbuf0, agg_sh.at[pl.ds(row0 + rep * CHUNK, CHUNK)])
    pltpu.sync_copy(a_hbm, a_v)
    plsc.subcore_barrier()

    def ebase(j):
        return pl.multiple_of(w * EPW + j * CHUNK, 8)

    def load_idx(j, b):
        eb = ebase(j)
        pltpu.sync_copy(send_hbm.at[pl.ds(eb, CHUNK)], idx_s[b])
        pltpu.sync_copy(recv_hbm.at[pl.ds(eb, CHUNK)], idx_r[b])

    def issue(j, b):
        pltpu.async_copy(p_hbm.at[idx_s[b]], gbuf[b], sem_g[b])
        pltpu.async_copy(eproj_hbm.at[pl.ds(ebase(j), CHUNK)],
                         pbuf[b], sem_e[b])

    avs = tuple(a_v[pl.ds(c8 * L, L)] for c8 in range(HIDDEN // L))

    def compute(b):
        g, p = gbuf[b], pbuf[b]

        @pl.loop(0, CHUNK)
        def _row(t):
            for c8 in range(HIDDEN // L):
                sl = pl.ds(c8 * L, L)
                v = g[t, sl] + p[t, sl]
                p[t, sl] = jnp.where(v >= 0.0, v, v * avs[c8])

    def step(j, b, nextj=None):
        eb = ebase(j)
        pltpu.make_async_copy(eproj_hbm.at[pl.ds(eb, CHUNK)],
                              pbuf[b], sem_e[b]).wait()
        pltpu.make_async_copy(p_hbm.at[idx_s[b]], gbuf[b], sem_g[b]).wait()
        compute(b)
        desc_s = pltpu.async_copy(pbuf[b], ue_hbm.at[pl.ds(eb, CHUNK)],
                                  sem_o[b])
        pltpu.sync_copy(pbuf[b], agg_sh.at[idx_r[b]], add=True)
        desc_s.wait()
        if nextj is not None:
            load_idx(nextj, b)
            issue(nextj, b)

    load_idx(0, 0)
    issue(0, 0)
    load_idx(1, 1)
    issue(1, 1)

    @pl.loop(0, CPW - 4, step=2)
    def _pair(j0):
        step(j0, 0, j0 + 2)
        step(j0 + 1, 1, j0 + 3)

    step(CPW - 3, 0, CPW - 1)
    step(CPW - 2, 1)
    step(CPW - 1, 0)

    plsc.subcore_barrier()
    pltpu.sync_copy(agg_sh.at[pl.ds(row0, ROWS_PER_TILE)],
                    agg_hbm.at[c, pl.ds(row0, ROWS_PER_TILE)])


_sc_edge = pl.kernel(
    _sc_edge_body,
    out_type=(
        jax.ShapeDtypeStruct((N_EDGES, HIDDEN), jnp.float32),
        jax.ShapeDtypeStruct((NC, N_PAD, HIDDEN), jnp.float32),
    ),
    mesh=_sc_mesh,
    scratch_types=(
        [pltpu.VMEM((CHUNK,), jnp.int32)] * 4
        + [pltpu.VMEM((CHUNK, HIDDEN), jnp.float32)] * 4
        + [pltpu.VMEM((HIDDEN,), jnp.float32),
           pltpu.VMEM_SHARED((N_PAD, HIDDEN), jnp.float32)]
        + [pltpu.SemaphoreType.DMA] * 6
    ),
)


def _matmul_body(x_ref, w_ref, o_ref):
    o_ref[...] = jnp.dot(x_ref[...], w_ref[...],
                         preferred_element_type=jnp.float32)


def _node_proj(x, w_t):
    blk = 2000
    return pl.pallas_call(
        _matmul_body,
        grid=(N_NODES // blk,),
        in_specs=[pl.BlockSpec((blk, D_NODE), lambda i: (i, 0)),
                  pl.BlockSpec((D_NODE, HIDDEN), lambda i: (0, 0))],
        out_specs=pl.BlockSpec((blk, HIDDEN), lambda i: (i, 0)),
        out_shape=jax.ShapeDtypeStruct((N_NODES, HIDDEN), jnp.float32),
    )(x, w_t)


def _edge_proj(ef, w_t):
    blk = 4000
    return pl.pallas_call(
        _matmul_body,
        grid=(N_EDGES // blk,),
        in_specs=[pl.BlockSpec((blk, D_EDGE), lambda i: (i, 0)),
                  pl.BlockSpec((D_EDGE, HIDDEN), lambda i: (0, 0))],
        out_specs=pl.BlockSpec((blk, HIDDEN), lambda i: (i, 0)),
        out_shape=jax.ShapeDtypeStruct((N_EDGES, HIDDEN), jnp.float32),
    )(ef, w_t)


def _node_mlp_body(a0_ref, a1_ref, x_ref, wa_ref, wx_ref, an_ref, o_ref):
    acc = jnp.dot(a0_ref[...] + a1_ref[...], wa_ref[...],
                  preferred_element_type=jnp.float32)
    acc = acc + jnp.dot(x_ref[...], wx_ref[...],
                        preferred_element_type=jnp.float32)
    a = an_ref[...]
    o_ref[...] = jnp.where(acc >= 0.0, acc, acc * a)


def _node_mlp(agg0, agg1, x, wa_t, wx_t, a_n):
    blk = 2000
    return pl.pallas_call(
        _node_mlp_body,
        grid=(N_NODES // blk,),
        in_specs=[pl.BlockSpec((blk, HIDDEN), lambda i: (i, 0)),
                  pl.BlockSpec((blk, HIDDEN), lambda i: (i, 0)),
                  pl.BlockSpec((blk, D_NODE), lambda i: (i, 0)),
                  pl.BlockSpec((HIDDEN, HIDDEN), lambda i: (0, 0)),
                  pl.BlockSpec((D_NODE, HIDDEN), lambda i: (0, 0)),
                  pl.BlockSpec((1, HIDDEN), lambda i: (0, 0))],
        out_specs=pl.BlockSpec((blk, HIDDEN), lambda i: (i, 0)),
        out_shape=jax.ShapeDtypeStruct((N_NODES, HIDDEN), jnp.float32),
    )(agg0, agg1, x, wa_t, wx_t, a_n)


def kernel(node_features, edge_index, edge_features, W_e, a_e, W_n, a_n):
    receivers = edge_index[0]
    senders = edge_index[1]
    w_en_t = W_e[:, :D_NODE].T
    w_ee_t = W_e[:, D_NODE:].T
    w_na_t = W_n[:, :HIDDEN].T
    w_nx_t = W_n[:, HIDDEN:].T

    p = _node_proj(node_features, w_en_t)
    eproj = _edge_proj(edge_features, w_ee_t)

    updated_edge_features, agg_parts = _sc_edge(
        senders, receivers, p, eproj, a_e)

    updated_node_features = _node_mlp(
        agg_parts[0, :N_NODES], agg_parts[1, :N_NODES], node_features,
        w_na_t, w_nx_t, a_n.reshape(1, HIDDEN))
    return (updated_node_features, updated_edge_features)
